# baseline (speedup 1.0000x reference)
import functools
import math

import jax
import jax.numpy as jnp
from jax import lax
from jax.experimental import pallas as pl
from jax.experimental.pallas import tpu as pltpu
from jax.experimental.pallas import tpu_sc as plsc

D_MODEL = 64
LANES = 16
NC = 2
NS = 16
NW = NC * NS
CHUNK = 128
SCALE = math.sqrt(D_MODEL)


def _body(table, toks, bias, out, idx_v, rows_v, bias_v, gsem):
    n_chunks = toks.shape[1]
    wid = lax.axis_index("s") * NC + lax.axis_index("c")
    rows_per_w = n_chunks * CHUNK
    base = wid * rows_per_w

    pltpu.sync_copy(toks.at[wid], idx_v)
    pltpu.sync_copy(bias, bias_v)

    b_regs = [bias_v[pl.ds(k * LANES, LANES)] for k in range(D_MODEL // LANES)]

    @pl.loop(0, n_chunks)
    def _chunk(j):
        pltpu.async_copy(table.at[idx_v.at[j]], rows_v, gsem).wait()

        @pl.loop(0, CHUNK)
        def _row(r):
            for k in range(D_MODEL // LANES):
                sl = pl.ds(k * LANES, LANES)
                rows_v[r, sl] = rows_v[r, sl] * SCALE + b_regs[k]

        pltpu.sync_copy(rows_v, out.at[pl.ds(base + j * CHUNK, CHUNK)])


def kernel(tokens, embed_weights, embed_bias):
    n_tok = tokens.shape[0] * tokens.shape[1]
    rows_per_w = n_tok // NW
    n_chunks = rows_per_w // CHUNK
    toks3d = tokens.reshape(NW, n_chunks, CHUNK)

    mesh = plsc.VectorSubcoreMesh(
        core_axis_name="c", subcore_axis_name="s", num_cores=NC, num_subcores=NS
    )
    run = pl.kernel(
        _body,
        out_type=jax.ShapeDtypeStruct((n_tok, D_MODEL), jnp.float32),
        mesh=mesh,
        scratch_types=[
            pltpu.VMEM((n_chunks, CHUNK), jnp.int32),
            pltpu.VMEM((CHUNK, D_MODEL), jnp.float32),
            pltpu.VMEM((D_MODEL,), jnp.float32),
            pltpu.SemaphoreType.DMA,
        ],
        compiler_params=pltpu.CompilerParams(use_tc_tiling_on_sc=False),
    )
    out = run(embed_weights, toks3d, embed_bias)
    return out.reshape(tokens.shape[0], tokens.shape[1], D_MODEL)


# speedup vs baseline: 1.2334x; 1.2334x over previous
import math

import jax
import jax.numpy as jnp
from jax import lax
from jax.experimental import pallas as pl
from jax.experimental.pallas import tpu as pltpu
from jax.experimental.pallas import tpu_sc as plsc

D_MODEL = 64
LANES = 16
NC = 2
NS = 16
NW = NC * NS
CHUNK = 128
GPC = 4
SUP = CHUNK * GPC
SCALE = math.sqrt(D_MODEL)


def _body(table, toks, bias, out, idx_v, buf0, buf1, bias_v, gsem0, gsem1):
    n_chunks = toks.shape[1]
    n_sup = n_chunks // GPC
    bufs = (buf0, buf1)
    gsems = (gsem0, gsem1)
    wid = lax.axis_index("s") * NC + lax.axis_index("c")
    base = wid * n_chunks * CHUNK

    pltpu.sync_copy(toks.at[wid], idx_v)
    pltpu.sync_copy(bias, bias_v)
    b_regs = [bias_v[pl.ds(k * LANES, LANES)] for k in range(D_MODEL // LANES)]

    def fire(j, b):
        for i in range(GPC):
            pltpu.async_copy(
                table.at[idx_v.at[j * GPC + i]],
                bufs[b].at[pl.ds(i * CHUNK, CHUNK)],
                gsems[b],
            )

    def drain(j, b):
        for i in range(GPC):
            pltpu.make_async_copy(
                table.at[idx_v.at[j * GPC + i]],
                bufs[b].at[pl.ds(i * CHUNK, CHUNK)],
                gsems[b],
            ).wait()

    fire(0, 0)

    @pl.loop(0, n_sup // 2)
    def _outer(g):
        for b in range(2):
            j = g * 2 + b

            @pl.when(j + 1 < n_sup)
            def _():
                fire(j + 1, 1 - b)

            drain(j, b)

            @plsc.parallel_loop(0, SUP, unroll=8)
            def _row(r):
                for k in range(D_MODEL // LANES):
                    sl = pl.ds(k * LANES, LANES)
                    bufs[b][r, sl] = bufs[b][r, sl] * SCALE + b_regs[k]

            pltpu.sync_copy(bufs[b], out.at[pl.ds(base + j * SUP, SUP)])


def kernel(tokens, embed_weights, embed_bias):
    n_tok = tokens.shape[0] * tokens.shape[1]
    rows_per_w = n_tok // NW
    n_chunks = rows_per_w // CHUNK
    toks3d = tokens.reshape(NW, n_chunks, CHUNK)

    mesh = plsc.VectorSubcoreMesh(
        core_axis_name="c", subcore_axis_name="s", num_cores=NC, num_subcores=NS
    )
    run = pl.kernel(
        _body,
        out_type=jax.ShapeDtypeStruct((n_tok, D_MODEL), jnp.float32),
        mesh=mesh,
        scratch_types=[
            pltpu.VMEM((n_chunks, CHUNK), jnp.int32),
            pltpu.VMEM((SUP, D_MODEL), jnp.float32),
            pltpu.VMEM((SUP, D_MODEL), jnp.float32),
            pltpu.VMEM((D_MODEL,), jnp.float32),
            pltpu.SemaphoreType.DMA,
            pltpu.SemaphoreType.DMA,
        ],
        compiler_params=pltpu.CompilerParams(use_tc_tiling_on_sc=False),
    )
    out = run(embed_weights, toks3d, embed_bias)
    return out.reshape(tokens.shape[0], tokens.shape[1], D_MODEL)
